# baseline (speedup 1.0000x reference)
import jax
import jax.numpy as jnp
from jax.experimental import pallas as pl
from jax.experimental.pallas import tpu as pltpu

BATCH = 8
NUM_PATCHES = 784
D = 128
M = 16384
K_NN = 9
CHUNK = 2048
NUM_CHUNKS = M // CHUNK
NCAND = 16


def _nt_dot(a, b):
    return jax.lax.dot_general(a, b, (((1,), (1,)), ((), ())),
                               preferred_element_type=jnp.float32)


def _kernel(emb_ref, mb_ref, out_ref, mb2_ref, mbbf_ref, cand_ref):
    b = pl.program_id(0)

    @pl.when(b == 0)
    def _():
        mb = mb_ref[...]
        mb2_ref[...] = jnp.sum(mb * mb, axis=1, keepdims=True)
        mbbf_ref[...] = mb.astype(jnp.bfloat16)

    x = emb_ref[...]
    x2 = jnp.sum(x * x, axis=1)
    xbf = x.astype(jnp.bfloat16)

    def body(c, run_min):
        chunk = mbbf_ref[pl.ds(c * CHUNK, CHUNK), :]
        mb2 = mb2_ref[pl.ds(c * CHUNK, CHUNK), :]
        s = mb2 - 2.0 * _nt_dot(chunk, xbf)
        return jnp.minimum(run_min, jnp.min(s, axis=0, keepdims=True))

    init = jnp.full((1, NUM_PATCHES), jnp.inf, jnp.float32)
    smin = jax.lax.fori_loop(0, NUM_CHUNKS, body, init)

    cur = smin + x2.reshape(1, NUM_PATCHES)
    lane = jax.lax.broadcasted_iota(jnp.int32, (1, NUM_PATCHES), 1)
    for j in range(NCAND):
        pj = jnp.argmax(cur)
        cand_ref[pl.ds(b * NCAND + j, 1), :] = emb_ref[pl.ds(pj, 1), :]
        cur = jnp.where(lane == pj, -jnp.inf, cur)

    @pl.when(b == BATCH - 1)
    def _():
        mb2 = mb2_ref[...]

        cands = cand_ref[...]
        dp_c = mb2 - 2.0 * _nt_dot(mb_ref[...], cands)
        mnc = jnp.min(dp_c, axis=0, keepdims=True)
        f2c = jnp.sum(cands * cands, axis=1).reshape(1, BATCH * NCAND)
        mind2c = mnc + f2c
        feats = jnp.concatenate([
            cand_ref[pl.ds(
                i * NCAND + jnp.argmax(mind2c[0:1, i * NCAND:(i + 1) * NCAND]),
                1), :]
            for i in range(BATCH)], axis=0)

        ridx = jax.lax.broadcasted_iota(jnp.int32, (M, BATCH), 0)

        dpart = mb2 - 2.0 * _nt_dot(mb_ref[...], feats)
        mn_f = jnp.min(dpart, axis=0, keepdims=True)
        am_f = jnp.min(jnp.where(dpart == mn_f, ridx, M), axis=0,
                       keepdims=True)
        f2 = jnp.sum(feats * feats, axis=1).reshape(1, BATCH)
        score = jnp.sqrt(jnp.maximum(mn_f + f2, 1e-12))

        ns = jnp.concatenate(
            [mb_ref[pl.ds(am_f[0, i], 1), :] for i in range(BATCH)], axis=0)
        s = mb2 - 2.0 * _nt_dot(mb_ref[...], ns)
        vals = []
        for _ in range(K_NN):
            mn = jnp.min(s, axis=0, keepdims=True)
            am = jnp.min(jnp.where(s == mn, ridx, M), axis=0, keepdims=True)
            mask = ridx == am
            vals.append(
                jnp.sum(jnp.where(mask, dpart, 0.0), axis=0, keepdims=True))
            s = jnp.where(mask, jnp.inf, s)

        v = jnp.concatenate(vals, axis=0)
        d3 = jnp.sqrt(jnp.maximum(v + f2, 1e-12))
        e = jnp.exp(d3 - jnp.max(d3, axis=0, keepdims=True))
        w0 = 1.0 - e[0:1, :] / jnp.sum(e, axis=0, keepdims=True)
        out_ref[...] = w0 * score


@jax.jit
def kernel(embedding, memory_bank):
    pred = pl.pallas_call(
        _kernel,
        grid=(BATCH,),
        in_specs=[
            pl.BlockSpec((NUM_PATCHES, D), lambda b: (b, 0)),
            pl.BlockSpec((M, D), lambda b: (0, 0)),
        ],
        out_specs=pl.BlockSpec((1, BATCH), lambda b: (0, 0)),
        out_shape=jax.ShapeDtypeStruct((1, BATCH), jnp.float32),
        scratch_shapes=[
            pltpu.VMEM((M, 1), jnp.float32),
            pltpu.VMEM((M, D), jnp.bfloat16),
            pltpu.VMEM((BATCH * NCAND, D), jnp.float32),
        ],
    )(embedding, memory_bank)
    return pred.reshape(BATCH)


# speedup vs baseline: 1.0284x; 1.0284x over previous
import jax
import jax.numpy as jnp
from jax.experimental import pallas as pl
from jax.experimental.pallas import tpu as pltpu

BATCH = 8
NUM_PATCHES = 784
D = 128
M = 16384
K_NN = 9
CHUNK = 2048
NUM_CHUNKS = M // CHUNK
NCAND = 16


def _nt_dot(a, b):
    return jax.lax.dot_general(a, b, (((1,), (1,)), ((), ())),
                               preferred_element_type=jnp.float32)


def _kernel(emb_ref, mb_ref, out_ref, mb2_ref, mbbf_ref, cand_ref):
    b = pl.program_id(0)

    @pl.when(b == 0)
    def _():
        mb = mb_ref[...]
        mb2_ref[...] = jnp.sum(mb * mb, axis=1, keepdims=True)
        mbbf_ref[...] = mb.astype(jnp.bfloat16)

    x = emb_ref[...]
    x2 = jnp.sum(x * x, axis=1)
    xbf = x.astype(jnp.bfloat16)

    smin = jnp.full((1, NUM_PATCHES), jnp.inf, jnp.float32)
    for c in range(NUM_CHUNKS):
        chunk = mbbf_ref[pl.ds(c * CHUNK, CHUNK), :]
        mb2 = mb2_ref[pl.ds(c * CHUNK, CHUNK), :]
        s = mb2 - 2.0 * _nt_dot(chunk, xbf)
        smin = jnp.minimum(smin, jnp.min(s, axis=0, keepdims=True))

    cur = smin + x2.reshape(1, NUM_PATCHES)
    lane = jax.lax.broadcasted_iota(jnp.int32, (1, NUM_PATCHES), 1)
    for j in range(NCAND):
        pj = jnp.argmax(cur)
        cand_ref[pl.ds(b * NCAND + j, 1), :] = emb_ref[pl.ds(pj, 1), :]
        cur = jnp.where(lane == pj, -jnp.inf, cur)

    @pl.when(b == BATCH - 1)
    def _():
        mb2 = mb2_ref[...]

        cands = cand_ref[...]
        dp_c = mb2 - 2.0 * _nt_dot(mb_ref[...], cands)
        mnc = jnp.min(dp_c, axis=0, keepdims=True)
        f2c = jnp.sum(cands * cands, axis=1).reshape(1, BATCH * NCAND)
        mind2c = mnc + f2c
        feats = jnp.concatenate([
            cand_ref[pl.ds(
                i * NCAND + jnp.argmax(mind2c[0:1, i * NCAND:(i + 1) * NCAND]),
                1), :]
            for i in range(BATCH)], axis=0)

        ridx = jax.lax.broadcasted_iota(jnp.int32, (M, BATCH), 0)

        dpart = mb2 - 2.0 * _nt_dot(mb_ref[...], feats)
        mn_f = jnp.min(dpart, axis=0, keepdims=True)
        am_f = jnp.min(jnp.where(dpart == mn_f, ridx, M), axis=0,
                       keepdims=True)
        f2 = jnp.sum(feats * feats, axis=1).reshape(1, BATCH)
        score = jnp.sqrt(jnp.maximum(mn_f + f2, 1e-12))

        ns = jnp.concatenate(
            [mb_ref[pl.ds(am_f[0, i], 1), :] for i in range(BATCH)], axis=0)
        s = mb2 - 2.0 * _nt_dot(mb_ref[...], ns)
        vals = []
        for _ in range(K_NN):
            mn = jnp.min(s, axis=0, keepdims=True)
            am = jnp.min(jnp.where(s == mn, ridx, M), axis=0, keepdims=True)
            mask = ridx == am
            vals.append(
                jnp.sum(jnp.where(mask, dpart, 0.0), axis=0, keepdims=True))
            s = jnp.where(mask, jnp.inf, s)

        v = jnp.concatenate(vals, axis=0)
        d3 = jnp.sqrt(jnp.maximum(v + f2, 1e-12))
        e = jnp.exp(d3 - jnp.max(d3, axis=0, keepdims=True))
        w0 = 1.0 - e[0:1, :] / jnp.sum(e, axis=0, keepdims=True)
        out_ref[...] = w0 * score


@jax.jit
def kernel(embedding, memory_bank):
    pred = pl.pallas_call(
        _kernel,
        grid=(BATCH,),
        in_specs=[
            pl.BlockSpec((NUM_PATCHES, D), lambda b: (b, 0)),
            pl.BlockSpec((M, D), lambda b: (0, 0)),
        ],
        out_specs=pl.BlockSpec((1, BATCH), lambda b: (0, 0)),
        out_shape=jax.ShapeDtypeStruct((1, BATCH), jnp.float32),
        scratch_shapes=[
            pltpu.VMEM((M, 1), jnp.float32),
            pltpu.VMEM((M, D), jnp.bfloat16),
            pltpu.VMEM((BATCH * NCAND, D), jnp.float32),
        ],
    )(embedding, memory_bank)
    return pred.reshape(BATCH)


# speedup vs baseline: 1.2069x; 1.1736x over previous
import jax
import jax.numpy as jnp
from jax.experimental import pallas as pl
from jax.experimental.pallas import tpu as pltpu

BATCH = 8
NUM_PATCHES = 784
D = 128
M = 16384
K_NN = 9
CHUNK = 2048
NUM_CHUNKS = M // CHUNK
NCAND = 16


def _nt_dot(a, b):
    return jax.lax.dot_general(a, b, (((1,), (1,)), ((), ())),
                               preferred_element_type=jnp.float32)


def _kernel(emb_ref, mb_ref, out_ref, mb2_ref, cand_ref):
    b = pl.program_id(0)

    @pl.when(b == 0)
    def _():
        mb = mb_ref[...]
        mb2_ref[...] = jnp.sum(mb * mb, axis=1, keepdims=True)

    x = emb_ref[...]
    x2 = jnp.sum(x * x, axis=1)

    smin = jnp.full((1, NUM_PATCHES), jnp.inf, jnp.float32)
    for c in range(NUM_CHUNKS):
        chunk = mb_ref[pl.ds(c * CHUNK, CHUNK), :]
        mb2 = mb2_ref[pl.ds(c * CHUNK, CHUNK), :]
        s = mb2 - 2.0 * _nt_dot(chunk, x)
        smin = jnp.minimum(smin, jnp.min(s, axis=0, keepdims=True))

    mind2 = smin + x2.reshape(1, NUM_PATCHES)
    p = jnp.argmax(mind2)
    cand_ref[pl.ds(b, 1), :] = emb_ref[pl.ds(p, 1), :]

    @pl.when(b == BATCH - 1)
    def _():
        mb2 = mb2_ref[...]
        feats = cand_ref[...]
        ridx = jax.lax.broadcasted_iota(jnp.int32, (M, BATCH), 0)

        dpart = mb2 - 2.0 * _nt_dot(mb_ref[...], feats)
        mn_f = jnp.min(dpart, axis=0, keepdims=True)
        am_f = jnp.min(jnp.where(dpart == mn_f, ridx, M), axis=0,
                       keepdims=True)
        f2 = jnp.sum(feats * feats, axis=1).reshape(1, BATCH)
        score = jnp.sqrt(jnp.maximum(mn_f + f2, 1e-12))

        ns = jnp.concatenate(
            [mb_ref[pl.ds(am_f[0, i], 1), :] for i in range(BATCH)], axis=0)
        s = mb2 - 2.0 * _nt_dot(mb_ref[...], ns)
        vals = []
        for _ in range(K_NN):
            mn = jnp.min(s, axis=0, keepdims=True)
            am = jnp.min(jnp.where(s == mn, ridx, M), axis=0, keepdims=True)
            mask = ridx == am
            vals.append(
                jnp.sum(jnp.where(mask, dpart, 0.0), axis=0, keepdims=True))
            s = jnp.where(mask, jnp.inf, s)

        v = jnp.concatenate(vals, axis=0)
        d3 = jnp.sqrt(jnp.maximum(v + f2, 1e-12))
        e = jnp.exp(d3 - jnp.max(d3, axis=0, keepdims=True))
        w0 = 1.0 - e[0:1, :] / jnp.sum(e, axis=0, keepdims=True)
        out_ref[...] = w0 * score


@jax.jit
def kernel(embedding, memory_bank):
    pred = pl.pallas_call(
        _kernel,
        grid=(BATCH,),
        in_specs=[
            pl.BlockSpec((NUM_PATCHES, D), lambda b: (b, 0)),
            pl.BlockSpec((M, D), lambda b: (0, 0)),
        ],
        out_specs=pl.BlockSpec((1, BATCH), lambda b: (0, 0)),
        out_shape=jax.ShapeDtypeStruct((1, BATCH), jnp.float32),
        scratch_shapes=[
            pltpu.VMEM((M, 1), jnp.float32),
            pltpu.VMEM((BATCH, D), jnp.float32),
        ],
    )(embedding, memory_bank)
    return pred.reshape(BATCH)


# speedup vs baseline: 1.2073x; 1.0003x over previous
import jax
import jax.numpy as jnp
from jax.experimental import pallas as pl
from jax.experimental.pallas import tpu as pltpu

BATCH = 8
NUM_PATCHES = 784
D = 128
M = 16384
K_NN = 9
CHUNK = 4096
NUM_CHUNKS = M // CHUNK
NCAND = 16


def _nt_dot(a, b):
    return jax.lax.dot_general(a, b, (((1,), (1,)), ((), ())),
                               preferred_element_type=jnp.float32)


def _kernel(emb_ref, mb_ref, out_ref, mb2_ref, cand_ref):
    b = pl.program_id(0)

    @pl.when(b == 0)
    def _():
        mb = mb_ref[...]
        mb2_ref[...] = jnp.sum(mb * mb, axis=1, keepdims=True)

    x = emb_ref[...]
    x2 = jnp.sum(x * x, axis=1)

    smin = jnp.full((1, NUM_PATCHES), jnp.inf, jnp.float32)
    for c in range(NUM_CHUNKS):
        chunk = mb_ref[pl.ds(c * CHUNK, CHUNK), :]
        mb2 = mb2_ref[pl.ds(c * CHUNK, CHUNK), :]
        s = mb2 - 2.0 * _nt_dot(chunk, x)
        smin = jnp.minimum(smin, jnp.min(s, axis=0, keepdims=True))

    mind2 = smin + x2.reshape(1, NUM_PATCHES)
    p = jnp.argmax(mind2)
    cand_ref[pl.ds(b, 1), :] = emb_ref[pl.ds(p, 1), :]

    @pl.when(b == BATCH - 1)
    def _():
        mb2 = mb2_ref[...]
        feats = cand_ref[...]
        ridx = jax.lax.broadcasted_iota(jnp.int32, (M, BATCH), 0)

        dpart = mb2 - 2.0 * _nt_dot(mb_ref[...], feats)
        mn_f = jnp.min(dpart, axis=0, keepdims=True)
        am_f = jnp.min(jnp.where(dpart == mn_f, ridx, M), axis=0,
                       keepdims=True)
        f2 = jnp.sum(feats * feats, axis=1).reshape(1, BATCH)
        score = jnp.sqrt(jnp.maximum(mn_f + f2, 1e-12))

        ns = jnp.concatenate(
            [mb_ref[pl.ds(am_f[0, i], 1), :] for i in range(BATCH)], axis=0)
        s = mb2 - 2.0 * _nt_dot(mb_ref[...], ns)
        vals = []
        for _ in range(K_NN):
            mn = jnp.min(s, axis=0, keepdims=True)
            am = jnp.min(jnp.where(s == mn, ridx, M), axis=0, keepdims=True)
            mask = ridx == am
            vals.append(
                jnp.sum(jnp.where(mask, dpart, 0.0), axis=0, keepdims=True))
            s = jnp.where(mask, jnp.inf, s)

        v = jnp.concatenate(vals, axis=0)
        d3 = jnp.sqrt(jnp.maximum(v + f2, 1e-12))
        e = jnp.exp(d3 - jnp.max(d3, axis=0, keepdims=True))
        w0 = 1.0 - e[0:1, :] / jnp.sum(e, axis=0, keepdims=True)
        out_ref[...] = w0 * score


@jax.jit
def kernel(embedding, memory_bank):
    pred = pl.pallas_call(
        _kernel,
        grid=(BATCH,),
        in_specs=[
            pl.BlockSpec((NUM_PATCHES, D), lambda b: (b, 0)),
            pl.BlockSpec((M, D), lambda b: (0, 0)),
        ],
        out_specs=pl.BlockSpec((1, BATCH), lambda b: (0, 0)),
        out_shape=jax.ShapeDtypeStruct((1, BATCH), jnp.float32),
        scratch_shapes=[
            pltpu.VMEM((M, 1), jnp.float32),
            pltpu.VMEM((BATCH, D), jnp.float32),
        ],
    )(embedding, memory_bank)
    return pred.reshape(BATCH)


# speedup vs baseline: 2.0850x; 1.7270x over previous
import jax
import jax.numpy as jnp
from jax.experimental import pallas as pl
from jax.experimental.pallas import tpu as pltpu

BATCH = 8
NUM_PATCHES = 784
D = 128
M = 16384
K_NN = 9
CHUNK = 4096
NUM_CHUNKS = M // CHUNK
NCAND = 16


def _nt_dot(a, b):
    return jax.lax.dot_general(a, b, (((1,), (1,)), ((), ())),
                               preferred_element_type=jnp.float32)


def _kernel(emb_ref, mb_ref, out_ref, mb2_ref, cand_ref, dp_ref, st_ref):
    b = pl.program_id(0)

    @pl.when(b == 0)
    def _():
        mb = mb_ref[...]
        mb2_ref[...] = jnp.sum(mb * mb, axis=1, keepdims=True)

    x = emb_ref[...]
    x2 = jnp.sum(x * x, axis=1)

    smin = jnp.full((1, NUM_PATCHES), jnp.inf, jnp.float32)
    for c in range(NUM_CHUNKS):
        chunk = mb_ref[pl.ds(c * CHUNK, CHUNK), :]
        mb2 = mb2_ref[pl.ds(c * CHUNK, CHUNK), :]
        s = mb2 - 2.0 * _nt_dot(chunk, x)
        smin = jnp.minimum(smin, jnp.min(s, axis=0, keepdims=True))

    mind2 = smin + x2.reshape(1, NUM_PATCHES)
    p = jnp.argmax(mind2)
    cand_ref[pl.ds(b, 1), :] = emb_ref[pl.ds(p, 1), :]

    @pl.when(b == BATCH - 1)
    def _():
        mb2 = mb2_ref[...]
        feats = cand_ref[...]
        cidx = jax.lax.broadcasted_iota(jnp.int32, (BATCH, M), 1)

        dp_ref[...] = jnp.swapaxes(
            mb2 - 2.0 * _nt_dot(mb_ref[...], feats), 0, 1)
        dpart = dp_ref[...]
        mn_f = jnp.min(dpart, axis=1, keepdims=True)
        am_f = jnp.min(jnp.where(dpart == mn_f, cidx, M), axis=1,
                       keepdims=True)
        f2 = jnp.sum(feats * feats, axis=1, keepdims=True)
        score = jnp.sqrt(jnp.maximum(mn_f + f2, 1e-12))

        ns = jnp.concatenate(
            [mb_ref[pl.ds(am_f[i, 0], 1), :] for i in range(BATCH)], axis=0)
        st_ref[...] = jnp.swapaxes(
            mb2 - 2.0 * _nt_dot(mb_ref[...], ns), 0, 1)
        vals = []
        for _ in range(K_NN):
            s = st_ref[...]
            mn = jnp.min(s, axis=1, keepdims=True)
            am = jnp.min(jnp.where(s == mn, cidx, M), axis=1, keepdims=True)
            mask = cidx == am
            vals.append(
                jnp.sum(jnp.where(mask, dp_ref[...], 0.0), axis=1,
                        keepdims=True))
            st_ref[...] = jnp.where(mask, jnp.inf, s)

        v = jnp.concatenate(vals, axis=1)
        d3 = jnp.sqrt(jnp.maximum(v + f2, 1e-12))
        e = jnp.exp(d3 - jnp.max(d3, axis=1, keepdims=True))
        w0 = 1.0 - e[:, 0:1] / jnp.sum(e, axis=1, keepdims=True)
        out_ref[...] = w0 * score


@jax.jit
def kernel(embedding, memory_bank):
    pred = pl.pallas_call(
        _kernel,
        grid=(BATCH,),
        in_specs=[
            pl.BlockSpec((NUM_PATCHES, D), lambda b: (b, 0)),
            pl.BlockSpec((M, D), lambda b: (0, 0)),
        ],
        out_specs=pl.BlockSpec((BATCH, 1), lambda b: (0, 0)),
        out_shape=jax.ShapeDtypeStruct((BATCH, 1), jnp.float32),
        scratch_shapes=[
            pltpu.VMEM((M, 1), jnp.float32),
            pltpu.VMEM((BATCH, D), jnp.float32),
            pltpu.VMEM((BATCH, M), jnp.float32),
            pltpu.VMEM((BATCH, M), jnp.float32),
        ],
    )(embedding, memory_bank)
    return pred.reshape(BATCH)
